# baseline (speedup 1.0000x reference)
import functools

import jax
import jax.numpy as jnp
from jax import lax
from jax.experimental import pallas as pl
from jax.experimental.pallas import tpu as pltpu
from jax.experimental.pallas import tpu_sc as plsc

_NUM_USERS = 50000
_LATENT = 64
_R = 5
_N_LAYERS = 2
_NEG_SLOPE = 0.01
_MLP_BLK = 2000


def _leaky(x):
    return jnp.where(x > 0, x, _NEG_SLOPE * x)


def _att_mlp_body(x_ref, w1_ref, c_ref, w2_ref, b2_ref, o_ref):
    x = x_ref[...]
    h = jnp.dot(x, w1_ref[0], preferred_element_type=jnp.float32) + c_ref[0]
    h = _leaky(h)
    h = jnp.dot(h, w2_ref[0], preferred_element_type=jnp.float32) + b2_ref[0]
    o_ref[0] = h


def _att_mlp(x, w1a, c, w2, b2):
    n = x.shape[0]
    grid = (n // _MLP_BLK, _R)
    return pl.pallas_call(
        _att_mlp_body,
        grid=grid,
        in_specs=[
            pl.BlockSpec((_MLP_BLK, _LATENT), lambda i, r: (i, 0)),
            pl.BlockSpec((1, _LATENT, _LATENT), lambda i, r: (r, 0, 0)),
            pl.BlockSpec((1, 1, _LATENT), lambda i, r: (r, 0, 0)),
            pl.BlockSpec((1, _LATENT, _LATENT), lambda i, r: (r, 0, 0)),
            pl.BlockSpec((1, 1, _LATENT), lambda i, r: (r, 0, 0)),
        ],
        out_specs=pl.BlockSpec((1, _MLP_BLK, _LATENT), lambda i, r: (r, i, 0)),
        out_shape=jax.ShapeDtypeStruct((_R, n, _LATENT), jnp.float32),
    )(x, w1a, c, w2, b2)


def _pred_body(z_ref, w1_ref, b1_ref, w2_ref, b2_ref, w3_ref, b3_ref, o_ref):
    h = jnp.dot(z_ref[...], w1_ref[...], preferred_element_type=jnp.float32) + b1_ref[...]
    h = _leaky(h)
    h = jnp.dot(h, w2_ref[...], preferred_element_type=jnp.float32) + b2_ref[...]
    h = _leaky(h)
    o_ref[...] = jnp.dot(h, w3_ref[...], preferred_element_type=jnp.float32) + b3_ref[...]


def _pred(z, w1p, b1p, w2p, b2p, w3p, b3p):
    m = z.shape[0]
    return pl.pallas_call(
        _pred_body,
        out_shape=jax.ShapeDtypeStruct((m, 128), jnp.float32),
    )(z, w1p, b1p, w2p, b2p, w3p, b3p)


_NC = 2
_NS = 16
_K = 80
_NCHUNK = 4
_HALF = 50048
_STRIPE = _HALF // _NS


def _make_spmm(n_edges_total):
    per_w = n_edges_total // _NS
    assert per_w * _NS == n_edges_total and per_w % _K == 0
    niter = per_w // _K
    mesh = plsc.VectorSubcoreMesh(core_axis_name="c", subcore_axis_name="s")

    @functools.partial(
        pl.kernel,
        mesh=mesh,
        compiler_params=pltpu.CompilerParams(use_tc_tiling_on_sc=False),
        out_type=jax.ShapeDtypeStruct((_NC, _NCHUNK, _HALF, 16), jnp.float32),
        scratch_types=[
            pltpu.VMEM((2, _K), jnp.int32),
            pltpu.VMEM((_K,), jnp.int32),
            pltpu.VMEM((_K,), jnp.float32),
            pltpu.VMEM((_K, 16), jnp.float32),
            pltpu.VMEM((_STRIPE, 16), jnp.float32),
            pltpu.VMEM_SHARED((_HALF, 16), jnp.float32),
            pltpu.SemaphoreType.DMA,
        ],
    )
    def spmm(h0, h1, h2, h3, eidx, evals, out,
             ebuf, rowbuf, valbuf, msgbuf, zbuf, acc, sem):
        core = lax.axis_index("c")
        sub = lax.axis_index("s")

        def zrow(j, _):
            zbuf[j] = jnp.zeros((16,), jnp.float32)
            return 0
        lax.fori_loop(0, _STRIPE, zrow, 0)

        for c, h_c in enumerate((h0, h1, h2, h3)):
            pltpu.sync_copy(zbuf, acc.at[pl.ds(sub * _STRIPE, _STRIPE)])
            plsc.subcore_barrier()

            def step(i, _):
                pltpu.sync_copy(eidx.at[sub * niter + i], ebuf)
                pltpu.sync_copy(evals.at[pl.ds(sub * per_w + i * _K, _K)], valbuf)
                pltpu.async_copy(h_c.at[ebuf.at[0]], msgbuf, sem).wait()

                def scale16(g, _):
                    base = g * 16
                    vv = valbuf[pl.ds(base, 16)]
                    rv = ebuf[1, pl.ds(base, 16)] - core * _HALF
                    mask = (rv >= 0) & (rv < _HALF)
                    rowbuf[pl.ds(base, 16)] = jnp.where(mask, rv, 0)
                    vv = jnp.where(mask, vv, 0.0)
                    for j in range(16):
                        e = base + j
                        msgbuf[e] = msgbuf[e] * vv[j]
                    return 0
                lax.fori_loop(0, _K // 16, scale16, 0)
                pltpu.sync_copy(msgbuf, acc.at[rowbuf], add=True)
                return 0
            lax.fori_loop(0, niter, step, 0)
            plsc.subcore_barrier()
            pltpu.sync_copy(acc.at[pl.ds(sub * _STRIPE, _STRIPE)],
                            out.at[core, c, pl.ds(sub * _STRIPE, _STRIPE)])
            plsc.subcore_barrier()

    return spmm


def _spmm_mean(h, eidx, evals, n):
    hr = h.reshape(_R * n, _NCHUNK, 16).transpose(1, 0, 2)
    spmm = _make_spmm(eidx.shape[0] * _K)
    parts = spmm(hr[0], hr[1], hr[2], hr[3], eidx, evals)
    agg = jnp.concatenate([parts[0], parts[1][:, :n - _HALF, :]], axis=1)
    return agg.transpose(1, 0, 2).reshape(n, _LATENT)


def kernel(users, pos_items, neg_items, user_emb, item_emb, rating_emb,
           att_W1, att_b1, att_W2, att_b2,
           pred_W1, pred_b1, pred_W2, pred_b2, pred_W3, pred_b3,
           graph_rows, graph_cols, graph_vals):
    n = user_emb.shape[0] + item_emb.shape[0]

    w1a = att_W1[:, :_LATENT, :]
    w1b = att_W1[:, _LATENT:, :]
    t = rating_emb[1:_R + 1]
    c = (jnp.einsum("rk,rkj->rj", t, w1b) + att_b1)[:, None, :]
    b2 = att_b2[:, None, :]

    cols5 = (graph_cols.astype(jnp.int32)
             + (jnp.arange(_R, dtype=jnp.int32) * n)[:, None]).reshape(-1)
    rows5 = graph_rows.astype(jnp.int32).reshape(-1)
    eidx = jnp.stack([cols5.reshape(-1, _K), rows5.reshape(-1, _K)], axis=1)
    evals = (graph_vals / _R).reshape(-1)

    all_embs = jnp.concatenate([user_emb, item_emb], axis=0)
    acc = all_embs
    cur = all_embs
    for _ in range(_N_LAYERS):
        h = _att_mlp(cur, w1a, c, att_W2, b2)
        cur = _spmm_mean(h, eidx, evals, n)
        acc = acc + cur
    light = acc / (_N_LAYERS + 1)

    users_e = jnp.take(light, users, axis=0)
    pos_e = jnp.take(light, _NUM_USERS + pos_items, axis=0)
    neg_e = jnp.take(light, _NUM_USERS + neg_items, axis=0)

    w1p = jnp.pad(pred_W1, ((0, 0), (0, 64)))
    b1p = jnp.pad(pred_b1, (0, 64))[None, :]
    w2p = jnp.pad(pred_W2, ((0, 64), (0, 96)))
    b2p = jnp.pad(pred_b2, (0, 96))[None, :]
    w3p = jnp.pad(pred_W3, ((0, 96), (0, 127)))
    b3p = jnp.pad(pred_b3, (0, 127))[None, :]

    z = jnp.concatenate(
        [jnp.concatenate([users_e, users_e], axis=0),
         jnp.concatenate([pos_e, neg_e], axis=0)], axis=1)
    out = _pred(z, w1p, b1p, w2p, b2p, w3p, b3p)[:, 0]
    b = users.shape[0]
    return (out[:b], out[b:])


# speedup vs baseline: 1.7590x; 1.7590x over previous
import functools

import jax
import jax.numpy as jnp
from jax import lax
from jax.experimental import pallas as pl
from jax.experimental.pallas import tpu as pltpu
from jax.experimental.pallas import tpu_sc as plsc

_NUM_USERS = 50000
_LATENT = 64
_R = 5
_N_LAYERS = 2
_NEG_SLOPE = 0.01
_MLP_BLK = 2000


def _leaky(x):
    return jnp.where(x > 0, x, _NEG_SLOPE * x)


def _att_mlp_body(x_ref, w1_ref, c_ref, w2_ref, b2_ref, o_ref):
    x = x_ref[...]
    h = jnp.dot(x, w1_ref[0], preferred_element_type=jnp.float32) + c_ref[0]
    h = _leaky(h)
    h = jnp.dot(h, w2_ref[0], preferred_element_type=jnp.float32) + b2_ref[0]
    o_ref[0] = h


def _att_mlp(x, w1a, c, w2, b2):
    n = x.shape[0]
    grid = (n // _MLP_BLK, _R)
    return pl.pallas_call(
        _att_mlp_body,
        grid=grid,
        in_specs=[
            pl.BlockSpec((_MLP_BLK, _LATENT), lambda i, r: (i, 0)),
            pl.BlockSpec((1, _LATENT, _LATENT), lambda i, r: (r, 0, 0)),
            pl.BlockSpec((1, 1, _LATENT), lambda i, r: (r, 0, 0)),
            pl.BlockSpec((1, _LATENT, _LATENT), lambda i, r: (r, 0, 0)),
            pl.BlockSpec((1, 1, _LATENT), lambda i, r: (r, 0, 0)),
        ],
        out_specs=pl.BlockSpec((1, _MLP_BLK, _LATENT), lambda i, r: (r, i, 0)),
        out_shape=jax.ShapeDtypeStruct((_R, n, _LATENT), jnp.float32),
    )(x, w1a, c, w2, b2)


def _pred_body(z_ref, w1_ref, b1_ref, w2_ref, b2_ref, w3_ref, b3_ref, o_ref):
    h = jnp.dot(z_ref[...], w1_ref[...], preferred_element_type=jnp.float32) + b1_ref[...]
    h = _leaky(h)
    h = jnp.dot(h, w2_ref[...], preferred_element_type=jnp.float32) + b2_ref[...]
    h = _leaky(h)
    o_ref[...] = jnp.dot(h, w3_ref[...], preferred_element_type=jnp.float32) + b3_ref[...]


def _pred(z, w1p, b1p, w2p, b2p, w3p, b3p):
    m = z.shape[0]
    return pl.pallas_call(
        _pred_body,
        out_shape=jax.ShapeDtypeStruct((m, 128), jnp.float32),
    )(z, w1p, b1p, w2p, b2p, w3p, b3p)


_NC = 2
_NS = 16
_K = 80
_NCHUNK = 4
_HALF = 50048
_STRIPE = _HALF // _NS


_NDMA = 8
_SUP = _NDMA * _K


def _make_spmm(n_edges_total):
    per_w = n_edges_total // _NS
    assert per_w * _NS == n_edges_total and per_w % _SUP == 0
    niter = per_w // _SUP
    mesh = plsc.VectorSubcoreMesh(core_axis_name="c", subcore_axis_name="s")

    @functools.partial(
        pl.kernel,
        mesh=mesh,
        compiler_params=pltpu.CompilerParams(use_tc_tiling_on_sc=False),
        out_type=jax.ShapeDtypeStruct((_NC, _NCHUNK, _HALF, 16), jnp.float32),
        scratch_types=[
            pltpu.VMEM((2, _SUP), jnp.int32),
            pltpu.VMEM((_NDMA, _K), jnp.int32),
            pltpu.VMEM((_SUP,), jnp.float32),
            pltpu.VMEM((_SUP, 16), jnp.float32),
            pltpu.VMEM((_STRIPE, 16), jnp.float32),
            pltpu.VMEM_SHARED((_HALF, 16), jnp.float32),
            pltpu.SemaphoreType.DMA,
            pltpu.SemaphoreType.DMA,
        ],
    )
    def spmm(h0, h1, h2, h3, eidx, evals, out,
             ebuf, rowbuf, valbuf, msgbuf, zbuf, acc, gsem, ssem):
        core = lax.axis_index("c")
        sub = lax.axis_index("s")

        def zrow(j, _):
            zbuf[j] = jnp.zeros((16,), jnp.float32)
            return 0
        lax.fori_loop(0, _STRIPE, zrow, 0)

        for c, h_c in enumerate((h0, h1, h2, h3)):
            pltpu.sync_copy(zbuf, acc.at[pl.ds(sub * _STRIPE, _STRIPE)])
            plsc.subcore_barrier()

            def step(i, _):
                pltpu.sync_copy(eidx.at[sub * niter + i], ebuf)
                pltpu.sync_copy(evals.at[pl.ds(sub * per_w + i * _SUP, _SUP)], valbuf)
                gathers = [
                    pltpu.async_copy(h_c.at[ebuf.at[0, pl.ds(j * _K, _K)]],
                                     msgbuf.at[pl.ds(j * _K, _K)], gsem)
                    for j in range(_NDMA)
                ]
                for g in gathers:
                    g.wait()

                def scale16(g, _):
                    base = g * 16
                    vv = valbuf[pl.ds(base, 16)]
                    rv = ebuf[1, pl.ds(base, 16)] - core * _HALF
                    mask = (rv >= 0) & (rv < _HALF)
                    rowbuf[base // _K, pl.ds(base % _K, 16)] = jnp.where(mask, rv, 0)
                    vv = jnp.where(mask, vv, 0.0)
                    for j in range(16):
                        e = base + j
                        msgbuf[e] = msgbuf[e] * vv[j]
                    return 0
                lax.fori_loop(0, _SUP // 16, scale16, 0)
                scatters = [
                    pltpu.async_copy(msgbuf.at[pl.ds(j * _K, _K)],
                                     acc.at[rowbuf.at[j]], ssem, add=True)
                    for j in range(_NDMA)
                ]
                for s in scatters:
                    s.wait()
                return 0
            lax.fori_loop(0, niter, step, 0)
            plsc.subcore_barrier()
            pltpu.sync_copy(acc.at[pl.ds(sub * _STRIPE, _STRIPE)],
                            out.at[core, c, pl.ds(sub * _STRIPE, _STRIPE)])
            plsc.subcore_barrier()

    return spmm


def _spmm_mean(h, eidx, evals, n):
    hr = h.reshape(_R * n, _NCHUNK, 16).transpose(1, 0, 2)
    spmm = _make_spmm(eidx.shape[0] * _SUP)
    parts = spmm(hr[0], hr[1], hr[2], hr[3], eidx, evals)
    agg = jnp.concatenate([parts[0], parts[1][:, :n - _HALF, :]], axis=1)
    return agg.transpose(1, 0, 2).reshape(n, _LATENT)


def kernel(users, pos_items, neg_items, user_emb, item_emb, rating_emb,
           att_W1, att_b1, att_W2, att_b2,
           pred_W1, pred_b1, pred_W2, pred_b2, pred_W3, pred_b3,
           graph_rows, graph_cols, graph_vals):
    n = user_emb.shape[0] + item_emb.shape[0]

    w1a = att_W1[:, :_LATENT, :]
    w1b = att_W1[:, _LATENT:, :]
    t = rating_emb[1:_R + 1]
    c = (jnp.einsum("rk,rkj->rj", t, w1b) + att_b1)[:, None, :]
    b2 = att_b2[:, None, :]

    cols5 = (graph_cols.astype(jnp.int32)
             + (jnp.arange(_R, dtype=jnp.int32) * n)[:, None]).reshape(-1)
    rows5 = graph_rows.astype(jnp.int32).reshape(-1)
    evals = (graph_vals / _R).reshape(-1)
    ne = cols5.shape[0]
    epad = -ne % (_NS * _SUP)
    cols5 = jnp.pad(cols5, (0, epad))
    rows5 = jnp.pad(rows5, (0, epad))
    evals = jnp.pad(evals, (0, epad))
    eidx = jnp.stack([cols5.reshape(-1, _SUP), rows5.reshape(-1, _SUP)], axis=1)

    all_embs = jnp.concatenate([user_emb, item_emb], axis=0)
    acc = all_embs
    cur = all_embs
    for _ in range(_N_LAYERS):
        h = _att_mlp(cur, w1a, c, att_W2, b2)
        cur = _spmm_mean(h, eidx, evals, n)
        acc = acc + cur
    light = acc / (_N_LAYERS + 1)

    users_e = jnp.take(light, users, axis=0)
    pos_e = jnp.take(light, _NUM_USERS + pos_items, axis=0)
    neg_e = jnp.take(light, _NUM_USERS + neg_items, axis=0)

    w1p = jnp.pad(pred_W1, ((0, 0), (0, 64)))
    b1p = jnp.pad(pred_b1, (0, 64))[None, :]
    w2p = jnp.pad(pred_W2, ((0, 64), (0, 96)))
    b2p = jnp.pad(pred_b2, (0, 96))[None, :]
    w3p = jnp.pad(pred_W3, ((0, 96), (0, 127)))
    b3p = jnp.pad(pred_b3, (0, 127))[None, :]

    z = jnp.concatenate(
        [jnp.concatenate([users_e, users_e], axis=0),
         jnp.concatenate([pos_e, neg_e], axis=0)], axis=1)
    out = _pred(z, w1p, b1p, w2p, b2p, w3p, b3p)[:, 0]
    b = users.shape[0]
    return (out[:b], out[b:])
